# baseline (speedup 1.0000x reference)
import functools

import jax
import jax.numpy as jnp
from jax import lax
from jax.experimental import pallas as pl
from jax.experimental.pallas import tpu as pltpu
from jax.experimental.pallas import tpu_sc as plsc

NUM_CLASSES = 100000
NUM_DOMAINS = 1000
E = 128
BATCH = 16384

_info = plsc.get_sparse_core_info()
NC, NS, L = _info.num_cores, _info.num_subcores, _info.num_lanes
NW = NC * NS
ROWS_PER_W = BATCH // NW
CHUNK = 64
NCHUNK = ROWS_PER_W // CHUNK
NBUF = 3
GROUPS = CHUNK // 16
BLKS = E // 16
STAGE = 64


def _rsqrt16(x):
    i = lax.bitcast_convert_type(x, jnp.int32)
    i = jnp.int32(0x5F3759DF) - lax.shift_right_logical(i, 1)
    y = lax.bitcast_convert_type(i, jnp.float32)
    for _ in range(3):
        y = y * (jnp.float32(1.5) - jnp.float32(0.5) * x * y * y)
    return y


def _sc_partials(emb, gt_c, gt_d, cls_tab, dom_tab, off):
    mesh = plsc.VectorSubcoreMesh(core_axis_name="c", subcore_axis_name="s")

    @functools.partial(
        pl.kernel,
        mesh=mesh,
        out_type=jax.ShapeDtypeStruct((NW, L), jnp.float32),
        compiler_params=pltpu.CompilerParams(needs_layout_passes=False),
        scratch_types=[
            pltpu.VMEM((ROWS_PER_W,), jnp.int32),
            pltpu.VMEM((ROWS_PER_W,), jnp.int32),
            [pltpu.VMEM((CHUNK, E), jnp.float32) for _ in range(NBUF)],
            [pltpu.VMEM((CHUNK, E), jnp.float32) for _ in range(NBUF)],
            [pltpu.VMEM((CHUNK, E), jnp.float32) for _ in range(NBUF)],
            pltpu.VMEM((E,), jnp.float32),
            pltpu.VMEM((STAGE, E), jnp.float32),
            pltpu.VMEM_SHARED((NUM_DOMAINS, E), jnp.float32),
            pltpu.VMEM((L, CHUNK + 1), jnp.float32),
            pltpu.VMEM((L, CHUNK + 1), jnp.float32),
            pltpu.VMEM((L,), jnp.float32),
            [pltpu.SemaphoreType.DMA for _ in range(3 * NBUF)],
        ],
    )
    def k(emb_hbm, gtc_hbm, gtd_hbm, cls_hbm, dom_hbm, off_hbm, out_hbm,
          idxc_v, idxd_v, cls_b, dom_b, emb_b, off_v, stage_v, dom_sh,
          tt_v, st_v, acc_v, sems):
        cid = lax.axis_index("c")
        sid = lax.axis_index("s")
        wid = sid * NC + cid
        row0 = wid * ROWS_PER_W

        def start_hbm(c):
            b = c % NBUF
            return (
                pltpu.async_copy(
                    cls_hbm.at[idxc_v.at[pl.ds(c * CHUNK, CHUNK)]],
                    cls_b[b], sems[3 * b + 0]),
                pltpu.async_copy(
                    emb_hbm.at[pl.ds(row0 + c * CHUNK, CHUNK)],
                    emb_b[b], sems[3 * b + 1]),
            )

        def start_dom(c):
            b = c % NBUF
            return pltpu.async_copy(
                dom_sh.at[idxd_v.at[pl.ds(c * CHUNK, CHUNK)]],
                dom_b[b], sems[3 * b + 2])

        pltpu.sync_copy(gtc_hbm.at[pl.ds(row0, ROWS_PER_W)], idxc_v)
        pltpu.sync_copy(gtd_hbm.at[pl.ds(row0, ROWS_PER_W)], idxd_v)
        pend_hbm = {c: start_hbm(c) for c in range(NBUF - 1)}

        pltpu.sync_copy(off_hbm, off_v)
        dbase = jnp.minimum(sid * STAGE, NUM_DOMAINS - STAGE)
        pltpu.sync_copy(dom_hbm.at[pl.ds(dbase, STAGE)], stage_v)

        def add_off(r, carry):
            for t in range(BLKS):
                sl = pl.ds(16 * t, 16)
                stage_v[r, sl] = stage_v[r, sl] + off_v[sl]
            return carry

        lax.fori_loop(0, STAGE, add_off, 0)
        pltpu.sync_copy(stage_v, dom_sh.at[pl.ds(dbase, STAGE)])
        plsc.subcore_barrier()
        pend_dom = {c: start_dom(c) for c in range(NBUF - 1)}

        lane = lax.iota(jnp.int32, L)
        z = jnp.zeros((L,), jnp.float32)
        accP = z
        accTS = z
        for c in range(NCHUNK):
            if c + NBUF - 1 < NCHUNK:
                pend_hbm[c + NBUF - 1] = start_hbm(c + NBUF - 1)
                pend_dom[c + NBUF - 1] = start_dom(c + NBUF - 1)
            for cp in pend_hbm.pop(c):
                cp.wait()
            pend_dom.pop(c).wait()
            b = c % NBUF
            cls_v, dom_v, emb_v = cls_b[b], dom_b[b], emb_b[b]

            def rowA(r, aP):
                rvec = jnp.zeros((L,), jnp.int32) + r
                sl0 = pl.ds(0, 16)
                e = emb_v[r, sl0]
                pred = cls_v[r, sl0] + dom_v[r, sl0]
                p0 = pred * pred
                t0 = pred * e
                s0 = e * e
                sl1 = pl.ds(16, 16)
                e = emb_v[r, sl1]
                pred = cls_v[r, sl1] + dom_v[r, sl1]
                p1 = pred * pred
                t1 = pred * e
                s1 = e * e
                for jj in range(2, BLKS):
                    sl = pl.ds(16 * jj, 16)
                    e = emb_v[r, sl]
                    pred = cls_v[r, sl] + dom_v[r, sl]
                    if jj % 2 == 0:
                        p0 += pred * pred
                        t0 += pred * e
                        s0 += e * e
                    else:
                        p1 += pred * pred
                        t1 += pred * e
                        s1 += e * e
                plsc.store_scatter(tt_v, [lane, rvec], t0 + t1)
                plsc.store_scatter(st_v, [lane, rvec], s0 + s1)
                return aP + (p0 + p1)

            accP = plsc.parallel_loop(0, CHUNK, carry=accP)(rowA)

            for g in range(GROUPS):
                sl = pl.ds(g * 16, 16)
                t0 = tt_v[0, sl] + tt_v[1, sl]
                t1 = tt_v[2, sl] + tt_v[3, sl]
                s0 = st_v[0, sl] + st_v[1, sl]
                s1 = st_v[2, sl] + st_v[3, sl]
                for k2 in range(4, L, 4):
                    t0 += tt_v[k2, sl] + tt_v[k2 + 1, sl]
                    t1 += tt_v[k2 + 2, sl] + tt_v[k2 + 3, sl]
                    s0 += st_v[k2, sl] + st_v[k2 + 1, sl]
                    s1 += st_v[k2 + 2, sl] + st_v[k2 + 3, sl]
                T = t0 + t1
                S = s0 + s1
                accTS += T * _rsqrt16(S)
        acc_v[...] = accP - jnp.float32(2.0) * accTS
        pltpu.sync_copy(acc_v, out_hbm.at[wid])

    return k(emb, gt_c, gt_d, cls_tab, dom_tab, off)


def _finish(parts_ref, o_ref):
    s = jnp.sum(parts_ref[...]) * jnp.float32(1.0 / BATCH) + jnp.float32(1.0)
    o_ref[...] = jnp.full((1, 1), s, jnp.float32)


def kernel(embeddings, gt_classes, gt_domains, class_components,
           domain_components, offset_component):
    parts = _sc_partials(embeddings, gt_classes, gt_domains,
                         class_components, domain_components,
                         offset_component.reshape(E))
    out = pl.pallas_call(
        _finish,
        out_shape=jax.ShapeDtypeStruct((1, 1), jnp.float32),
    )(parts)
    return out[0, 0]


# speedup vs baseline: 1.3089x; 1.3089x over previous
import functools

import jax
import jax.numpy as jnp
from jax import lax
from jax.experimental import pallas as pl
from jax.experimental.pallas import tpu as pltpu
from jax.experimental.pallas import tpu_sc as plsc

NUM_CLASSES = 100000
NUM_DOMAINS = 1000
E = 128
BATCH = 16384

_info = plsc.get_sparse_core_info()
NC, NS, L = _info.num_cores, _info.num_subcores, _info.num_lanes
NW = NC * NS
ROWS_PER_W = BATCH // NW
CHUNK = 64
NCHUNK = ROWS_PER_W // CHUNK
NBUF = 3
GROUPS = CHUNK // 16
BLKS = E // 16
STAGE = 64


def _rsqrt16(x):
    i = lax.bitcast_convert_type(x, jnp.int32)
    i = jnp.int32(0x5F3759DF) - lax.shift_right_logical(i, 1)
    y = lax.bitcast_convert_type(i, jnp.float32)
    for _ in range(3):
        y = y * (jnp.float32(1.5) - jnp.float32(0.5) * x * y * y)
    return y


def _sc_partials(emb, gt_c, gt_d, cls_tab, dom_tab, off):
    mesh = plsc.VectorSubcoreMesh(core_axis_name="c", subcore_axis_name="s")

    @functools.partial(
        pl.kernel,
        mesh=mesh,
        out_type=jax.ShapeDtypeStruct((NW, L), jnp.float32),
        compiler_params=pltpu.CompilerParams(needs_layout_passes=False),
        scratch_types=[
            pltpu.VMEM((ROWS_PER_W,), jnp.int32),
            pltpu.VMEM((ROWS_PER_W,), jnp.int32),
            [pltpu.VMEM((CHUNK, E), jnp.float32) for _ in range(NBUF)],
            [pltpu.VMEM((CHUNK, E), jnp.float32) for _ in range(NBUF)],
            [pltpu.VMEM((CHUNK, E), jnp.float32) for _ in range(NBUF)],
            pltpu.VMEM((E,), jnp.float32),
            pltpu.VMEM((STAGE, E), jnp.float32),
            pltpu.VMEM_SHARED((NUM_DOMAINS, E), jnp.float32),
            pltpu.VMEM((L,), jnp.float32),
            [pltpu.SemaphoreType.DMA for _ in range(3 * NBUF)],
        ],
    )
    def k(emb_hbm, gtc_hbm, gtd_hbm, cls_hbm, dom_hbm, off_hbm, out_hbm,
          idxc_v, idxd_v, cls_b, dom_b, emb_b, off_v, stage_v, dom_sh,
          acc_v, sems):
        cid = lax.axis_index("c")
        sid = lax.axis_index("s")
        wid = sid * NC + cid
        row0 = wid * ROWS_PER_W

        def start_hbm(c):
            b = c % NBUF
            return (
                pltpu.async_copy(
                    cls_hbm.at[idxc_v.at[pl.ds(c * CHUNK, CHUNK)]],
                    cls_b[b], sems[3 * b + 0]),
                pltpu.async_copy(
                    emb_hbm.at[pl.ds(row0 + c * CHUNK, CHUNK)],
                    emb_b[b], sems[3 * b + 1]),
            )

        def start_dom(c):
            b = c % NBUF
            return pltpu.async_copy(
                dom_sh.at[idxd_v.at[pl.ds(c * CHUNK, CHUNK)]],
                dom_b[b], sems[3 * b + 2])

        pltpu.sync_copy(gtc_hbm.at[pl.ds(row0, ROWS_PER_W)], idxc_v)
        pltpu.sync_copy(gtd_hbm.at[pl.ds(row0, ROWS_PER_W)], idxd_v)
        pend_hbm = {c: start_hbm(c) for c in range(NBUF - 1)}

        pltpu.sync_copy(off_hbm, off_v)
        dbase = jnp.minimum(sid * STAGE, NUM_DOMAINS - STAGE)
        pltpu.sync_copy(dom_hbm.at[pl.ds(dbase, STAGE)], stage_v)

        def add_off(r, carry):
            for t in range(BLKS):
                sl = pl.ds(16 * t, 16)
                stage_v[r, sl] = stage_v[r, sl] + off_v[sl]
            return carry

        lax.fori_loop(0, STAGE, add_off, 0)
        pltpu.sync_copy(stage_v, dom_sh.at[pl.ds(dbase, STAGE)])
        plsc.subcore_barrier()
        pend_dom = {c: start_dom(c) for c in range(NBUF - 1)}

        lane = lax.iota(jnp.int32, L)
        z = jnp.zeros((L,), jnp.float32)
        perm = [(lane + u) & jnp.int32(L - 1) for u in range(16)]
        acc = z
        for c in range(NCHUNK):
            if c + NBUF - 1 < NCHUNK:
                pend_hbm[c + NBUF - 1] = start_hbm(c + NBUF - 1)
                pend_dom[c + NBUF - 1] = start_dom(c + NBUF - 1)
            for cp in pend_hbm.pop(c):
                cp.wait()
            pend_dom.pop(c).wait()
            b = c % NBUF
            cls_v, dom_v, emb_v = cls_b[b], dom_b[b], emb_b[b]

            def group(g, a):
                rows = g * 16 + lane

                def blk(jj, carry):
                    P0, T0, S0, P1, T1, S1 = carry
                    jj16 = jj * 16
                    for u in range(16):
                        cj = perm[u] + jj16
                        e = plsc.load_gather(emb_v, [rows, cj])
                        cc = plsc.load_gather(cls_v, [rows, cj])
                        dd = plsc.load_gather(dom_v, [rows, cj])
                        pred = cc + dd
                        if u % 2 == 0:
                            P0 += pred * pred
                            T0 += pred * e
                            S0 += e * e
                        else:
                            P1 += pred * pred
                            T1 += pred * e
                            S1 += e * e
                    return (P0, T0, S0, P1, T1, S1)

                P0, T0, S0, P1, T1, S1 = plsc.parallel_loop(
                    0, BLKS, carry=(z, z, z, z, z, z))(blk)
                P, T, S = P0 + P1, T0 + T1, S0 + S1
                return a + P - jnp.float32(2.0) * T * _rsqrt16(S)

            acc = plsc.parallel_loop(0, GROUPS, carry=acc)(group)
        acc_v[...] = acc
        pltpu.sync_copy(acc_v, out_hbm.at[wid])

    return k(emb, gt_c, gt_d, cls_tab, dom_tab, off)


def _finish(parts_ref, o_ref):
    s = jnp.sum(parts_ref[...]) * jnp.float32(1.0 / BATCH) + jnp.float32(1.0)
    o_ref[...] = jnp.full((1, 1), s, jnp.float32)


def kernel(embeddings, gt_classes, gt_domains, class_components,
           domain_components, offset_component):
    parts = _sc_partials(embeddings, gt_classes, gt_domains,
                         class_components, domain_components,
                         offset_component.reshape(E))
    out = pl.pallas_call(
        _finish,
        out_shape=jax.ShapeDtypeStruct((1, 1), jnp.float32),
    )(parts)
    return out[0, 0]


# speedup vs baseline: 1.3493x; 1.0309x over previous
import functools

import jax
import jax.numpy as jnp
from jax import lax
from jax.experimental import pallas as pl
from jax.experimental.pallas import tpu as pltpu
from jax.experimental.pallas import tpu_sc as plsc

NUM_CLASSES = 100000
NUM_DOMAINS = 1000
E = 128
BATCH = 16384

_info = plsc.get_sparse_core_info()
NC, NS, L = _info.num_cores, _info.num_subcores, _info.num_lanes
NW = NC * NS
ROWS_PER_W = BATCH // NW
CHUNK = 128
NCHUNK = ROWS_PER_W // CHUNK
NBUF = 2
GROUPS = CHUNK // 16
BLKS = E // 16
STAGE = 64


def _rsqrt16(x):
    i = lax.bitcast_convert_type(x, jnp.int32)
    i = jnp.int32(0x5F3759DF) - lax.shift_right_logical(i, 1)
    y = lax.bitcast_convert_type(i, jnp.float32)
    for _ in range(3):
        y = y * (jnp.float32(1.5) - jnp.float32(0.5) * x * y * y)
    return y


def _sc_partials(emb, gt_c, gt_d, cls_tab, dom_tab, off):
    mesh = plsc.VectorSubcoreMesh(core_axis_name="c", subcore_axis_name="s")

    @functools.partial(
        pl.kernel,
        mesh=mesh,
        out_type=jax.ShapeDtypeStruct((NW, L), jnp.float32),
        compiler_params=pltpu.CompilerParams(needs_layout_passes=False),
        scratch_types=[
            pltpu.VMEM((ROWS_PER_W,), jnp.int32),
            pltpu.VMEM((ROWS_PER_W,), jnp.int32),
            [pltpu.VMEM((CHUNK, E), jnp.float32) for _ in range(NBUF)],
            [pltpu.VMEM((CHUNK, E), jnp.float32) for _ in range(NBUF)],
            [pltpu.VMEM((CHUNK, E), jnp.float32) for _ in range(NBUF)],
            pltpu.VMEM((E,), jnp.float32),
            pltpu.VMEM((STAGE, E), jnp.float32),
            pltpu.VMEM_SHARED((NUM_DOMAINS, E), jnp.float32),
            pltpu.VMEM((L,), jnp.float32),
            [pltpu.SemaphoreType.DMA for _ in range(3 * NBUF)],
        ],
    )
    def k(emb_hbm, gtc_hbm, gtd_hbm, cls_hbm, dom_hbm, off_hbm, out_hbm,
          idxc_v, idxd_v, cls_b, dom_b, emb_b, off_v, stage_v, dom_sh,
          acc_v, sems):
        cid = lax.axis_index("c")
        sid = lax.axis_index("s")
        wid = sid * NC + cid
        row0 = wid * ROWS_PER_W

        def start_hbm(c):
            b = c % NBUF
            return (
                pltpu.async_copy(
                    cls_hbm.at[idxc_v.at[pl.ds(c * CHUNK, CHUNK)]],
                    cls_b[b], sems[3 * b + 0]),
                pltpu.async_copy(
                    emb_hbm.at[pl.ds(row0 + c * CHUNK, CHUNK)],
                    emb_b[b], sems[3 * b + 1]),
            )

        def start_dom(c):
            b = c % NBUF
            return pltpu.async_copy(
                dom_sh.at[idxd_v.at[pl.ds(c * CHUNK, CHUNK)]],
                dom_b[b], sems[3 * b + 2])

        pltpu.sync_copy(gtc_hbm.at[pl.ds(row0, ROWS_PER_W)], idxc_v)
        pltpu.sync_copy(gtd_hbm.at[pl.ds(row0, ROWS_PER_W)], idxd_v)
        pend_hbm = {c: start_hbm(c) for c in range(NBUF - 1)}

        pltpu.sync_copy(off_hbm, off_v)
        dbase = jnp.minimum(sid * STAGE, NUM_DOMAINS - STAGE)
        pltpu.sync_copy(dom_hbm.at[pl.ds(dbase, STAGE)], stage_v)

        def add_off(r, carry):
            for t in range(BLKS):
                sl = pl.ds(16 * t, 16)
                stage_v[r, sl] = stage_v[r, sl] + off_v[sl]
            return carry

        lax.fori_loop(0, STAGE, add_off, 0)
        pltpu.sync_copy(stage_v, dom_sh.at[pl.ds(dbase, STAGE)])
        plsc.subcore_barrier()
        pend_dom = {c: start_dom(c) for c in range(NBUF - 1)}

        lane = lax.iota(jnp.int32, L)
        z = jnp.zeros((L,), jnp.float32)
        perm = [(lane + u) & jnp.int32(L - 1) for u in range(16)]
        acc = z
        for c in range(NCHUNK):
            if c + NBUF - 1 < NCHUNK:
                pend_hbm[c + NBUF - 1] = start_hbm(c + NBUF - 1)
                pend_dom[c + NBUF - 1] = start_dom(c + NBUF - 1)
            for cp in pend_hbm.pop(c):
                cp.wait()
            pend_dom.pop(c).wait()
            b = c % NBUF
            cls_v, dom_v, emb_v = cls_b[b], dom_b[b], emb_b[b]

            def group(g, a):
                rows = g * 16 + lane

                def blk(jj, carry):
                    P0, T0, S0, P1, T1, S1 = carry
                    jj16 = jj * 16
                    for u in range(16):
                        cj = perm[u] + jj16
                        e = plsc.load_gather(emb_v, [rows, cj])
                        cc = plsc.load_gather(cls_v, [rows, cj])
                        dd = plsc.load_gather(dom_v, [rows, cj])
                        pred = cc + dd
                        if u % 2 == 0:
                            P0 += pred * pred
                            T0 += pred * e
                            S0 += e * e
                        else:
                            P1 += pred * pred
                            T1 += pred * e
                            S1 += e * e
                    return (P0, T0, S0, P1, T1, S1)

                P0, T0, S0, P1, T1, S1 = plsc.parallel_loop(
                    0, BLKS, carry=(z, z, z, z, z, z))(blk)
                P, T, S = P0 + P1, T0 + T1, S0 + S1
                return a + P - jnp.float32(2.0) * T * _rsqrt16(S)

            acc = plsc.parallel_loop(0, GROUPS, carry=acc)(group)
        acc_v[...] = acc
        pltpu.sync_copy(acc_v, out_hbm.at[wid])

    return k(emb, gt_c, gt_d, cls_tab, dom_tab, off)


def _finish(parts_ref, o_ref):
    s = jnp.sum(parts_ref[...]) * jnp.float32(1.0 / BATCH) + jnp.float32(1.0)
    o_ref[...] = jnp.full((1, 1), s, jnp.float32)


def kernel(embeddings, gt_classes, gt_domains, class_components,
           domain_components, offset_component):
    parts = _sc_partials(embeddings, gt_classes, gt_domains,
                         class_components, domain_components,
                         offset_component.reshape(E))
    out = pl.pallas_call(
        _finish,
        out_shape=jax.ShapeDtypeStruct((1, 1), jnp.float32),
    )(parts)
    return out[0, 0]
